# baseline (speedup 1.0000x reference)
import functools

import jax
import jax.numpy as jnp
from jax import lax
from jax.experimental import pallas as pl
from jax.experimental.pallas import tpu as pltpu
from jax.experimental.pallas import tpu_sc as plsc

NSC, NSUB = 2, 16
NW = NSC * NSUB
NPT = 320
NROW = NW * NPT
NPZ = 10368
ZSTRIPE = NPZ // NSUB


def _mesh():
    return plsc.VectorSubcoreMesh(core_axis_name="c", subcore_axis_name="s")


def _wid():
    return lax.axis_index("s") * NSC + lax.axis_index("c")


def _iota16():
    return lax.iota(jnp.int32, 16)


def _dyn_gather(x, idx):
    return x.at[idx].get(mode="promise_in_bounds")


def _phase0(srcr, dstr, E):
    nrows = E // 128
    SR = 20
    nchunks = nrows // SR
    ECAP = E + 128

    def body(srcr_h, dstr_h, bsrc_h, bdst_h, bk_h, cnt_h,
             sv, dv, st_s, st_d, st_k, csm):
        w = _wid()
        lo = w * NPT
        iota = _iota16()

        def chunk_body(ci, carry):
            cur, out = carry
            rbase = ci * SR
            pltpu.sync_copy(srcr_h.at[pl.ds(rbase, SR), :], sv)
            pltpu.sync_copy(dstr_h.at[pl.ds(rbase, SR), :], dv)

            def group_body(g, carry2):
                cur2, = carry2
                j = g // 8
                sub = g % 8
                d16 = dv[j, pl.ds(sub * 16, 16)]
                s16 = sv[j, pl.ds(sub * 16, 16)]
                k16 = (ci * (SR * 128) + g * 16) + iota
                msk = (d16 >= lo) & (d16 < lo + NPT)
                plsc.store_compressed(st_s.at[pl.ds(cur2, 16)], s16, mask=msk)
                plsc.store_compressed(st_d.at[pl.ds(cur2, 16)], d16 - lo,
                                      mask=msk)
                plsc.store_compressed(st_k.at[pl.ds(cur2, 16)], k16, mask=msk)
                cnt = jnp.sum(msk.astype(jnp.int32))
                return (cur2 + cnt,)

            cur, = lax.fori_loop(0, SR * 8, group_body, (cur,))

            nfl = cur // 128

            def flush_body(j, _):
                o = pl.multiple_of(out + j * 128, 128)
                pltpu.sync_copy(st_s.at[pl.ds(j * 128, 128)],
                                bsrc_h.at[w, pl.ds(o, 128)])
                pltpu.sync_copy(st_d.at[pl.ds(j * 128, 128)],
                                bdst_h.at[w, pl.ds(o, 128)])
                pltpu.sync_copy(st_k.at[pl.ds(j * 128, 128)],
                                bk_h.at[w, pl.ds(o, 128)])
                return 0

            lax.fori_loop(0, nfl, flush_body, 0)
            tb = nfl * 128
            for i in range(8):
                st_s[pl.ds(i * 16, 16)] = st_s[pl.ds(tb + i * 16, 16)]
                st_d[pl.ds(i * 16, 16)] = st_d[pl.ds(tb + i * 16, 16)]
                st_k[pl.ds(i * 16, 16)] = st_k[pl.ds(tb + i * 16, 16)]
            return (cur - tb, out + tb)

        cur, out = lax.fori_loop(0, nchunks, chunk_body, (0, 0))

        @pl.when(cur > 0)
        def _():
            for i in range(8):
                st_s[pl.ds(cur + i * 16, 16)] = jnp.zeros(16, jnp.int32)
                st_d[pl.ds(cur + i * 16, 16)] = jnp.full(16, NPT, jnp.int32)
                st_k[pl.ds(cur + i * 16, 16)] = jnp.zeros(16, jnp.int32)
            o = pl.multiple_of(out, 128)
            pltpu.sync_copy(st_s.at[pl.ds(0, 128)],
                            bsrc_h.at[w, pl.ds(o, 128)])
            pltpu.sync_copy(st_d.at[pl.ds(0, 128)],
                            bdst_h.at[w, pl.ds(o, 128)])
            pltpu.sync_copy(st_k.at[pl.ds(0, 128)],
                            bk_h.at[w, pl.ds(o, 128)])

        total = jnp.where(cur > 0, out + 128, out)
        csm[:] = jnp.full(16, total, jnp.int32)
        pltpu.sync_copy(csm, cnt_h.at[w, :])

    f = pl.kernel(
        body,
        out_type=(
            jax.ShapeDtypeStruct((NW, ECAP), jnp.int32),
            jax.ShapeDtypeStruct((NW, ECAP), jnp.int32),
            jax.ShapeDtypeStruct((NW, ECAP), jnp.int32),
            jax.ShapeDtypeStruct((NW, 16), jnp.int32),
        ),
        mesh=_mesh(),
        compiler_params=pltpu.CompilerParams(use_tc_tiling_on_sc=False, needs_layout_passes=False),
        scratch_types=[
            pltpu.VMEM((SR, 128), jnp.int32),
            pltpu.VMEM((SR, 128), jnp.int32),
            pltpu.VMEM((2704,), jnp.int32),
            pltpu.VMEM((2704,), jnp.int32),
            pltpu.VMEM((2704,), jnp.int32),
            pltpu.VMEM((16,), jnp.int32),
        ],
    )
    return f(srcr, dstr)


def _phase1(a_cat, srcr, dstr, E):
    NBLK = E // 512
    npw = (NBLK + NW - 1) // NW

    def body(a_h, srcr_h, dstr_h, p_h, zp_h,
             sv, dv, as_r, ad_r, pb, zb, z_sh):
        w = _wid()
        cid = lax.axis_index("c")
        sid = lax.axis_index("s")
        iota = _iota16()

        def zp_body(r, _):
            pb[r, :] = jnp.zeros(16, jnp.float32)
            return 0
        lax.fori_loop(0, 512, zp_body, 0)

        def zz_body(r, _):
            zb[r, :] = jnp.zeros(16, jnp.float32)
            return 0
        lax.fori_loop(0, ZSTRIPE, zz_body, 0)
        pltpu.sync_copy(zb, z_sh.at[pl.ds(pl.multiple_of(sid * ZSTRIPE, 8),
                                          ZSTRIPE), :])
        plsc.subcore_barrier()

        def blk_body(j, _):
            b = j * NW + w

            @pl.when(b < NBLK)
            def _():
                rbase = b * 4
                pltpu.sync_copy(srcr_h.at[pl.ds(rbase, 4), :], sv)
                pltpu.sync_copy(dstr_h.at[pl.ds(rbase, 4), :], dv)
                for jj in range(4):
                    pltpu.sync_copy(a_h.at[sv.at[jj]],
                                    as_r.at[pl.ds(jj * 128, 128), :])
                    pltpu.sync_copy(a_h.at[dv.at[jj]],
                                    ad_r.at[pl.ds(jj * 128, 128), :])

                def grp_body(g, _2):
                    e16 = g * 16 + iota
                    for h in range(8):
                        fh = jnp.full(16, h, jnp.int32)
                        av = plsc.load_gather(as_r, [e16, fh])
                        bv = plsc.load_gather(ad_r, [e16, fh + 8])
                        ev = av + bv
                        lv = jnp.maximum(ev, 0.0) + 0.2 * jnp.minimum(ev, 0.0)
                        pv = jnp.exp(lv)
                        plsc.store_scatter(pb, [e16, fh], pv)
                    return 0

                lax.fori_loop(0, 32, grp_body, 0)
                pltpu.sync_copy(pb, p_h.at[pl.ds(pl.multiple_of(b * 512, 512), 512), :])
                for jj in range(4):
                    pltpu.sync_copy(pb.at[pl.ds(jj * 128, 128), :],
                                    z_sh.at[dv.at[jj]], add=True)
            return 0

        lax.fori_loop(0, npw, blk_body, 0)
        plsc.subcore_barrier()
        st8 = pl.multiple_of(sid * ZSTRIPE, 8)
        pltpu.sync_copy(z_sh.at[pl.ds(st8, ZSTRIPE), :],
                        zp_h.at[cid, pl.ds(st8, ZSTRIPE), :])

    f = pl.kernel(
        body,
        out_type=(
            jax.ShapeDtypeStruct((E, 16), jnp.float32),
            jax.ShapeDtypeStruct((NSC, NPZ, 16), jnp.float32),
        ),
        mesh=_mesh(),
        compiler_params=pltpu.CompilerParams(use_tc_tiling_on_sc=False, needs_layout_passes=False),
        scratch_types=[
            pltpu.VMEM((4, 128), jnp.int32),
            pltpu.VMEM((4, 128), jnp.int32),
            pltpu.VMEM((512, 16), jnp.float32),
            pltpu.VMEM((512, 16), jnp.float32),
            pltpu.VMEM((512, 16), jnp.float32),
            pltpu.VMEM((ZSTRIPE, 16), jnp.float32),
            pltpu.VMEM_SHARED((NPZ, 16), jnp.float32),
        ],
    )
    return f(a_cat, srcr, dstr)


def _phase1c(p, rz, bdst, bk, counts, E):
    ECAP = E + 128

    def body(p_h, rz_h, bdst_h, bk_h, cnt_h, ab_h,
             kb, dlb, dgb, prows, zrows, abuf, csm):
        w = _wid()
        iota = _iota16()
        pltpu.sync_copy(cnt_h.at[w, :], csm)
        nb = jnp.max(csm[...]) // 128

        def bat_body(jb, _):
            off = pl.multiple_of(jb * 128, 128)
            pltpu.sync_copy(bk_h.at[w, pl.ds(off, 128)], kb)
            pltpu.sync_copy(bdst_h.at[w, pl.ds(off, 128)], dlb)
            for i in range(8):
                dgb[pl.ds(i * 16, 16)] = dlb[pl.ds(i * 16, 16)] + w * NPT
            pltpu.sync_copy(p_h.at[kb], prows)
            pltpu.sync_copy(rz_h.at[dgb], zrows)
            for i in range(8):
                e16 = i * 16 + iota
                for h in range(8):
                    fh = jnp.full(16, h, jnp.int32)
                    av = (plsc.load_gather(prows, [e16, fh])
                          * plsc.load_gather(zrows, [e16, fh]))
                    plsc.store_scatter(abuf, [e16, fh], av)
            pltpu.sync_copy(abuf, ab_h.at[w, pl.ds(off, 128), :])
            return 0

        lax.fori_loop(0, nb, bat_body, 0)

    f = pl.kernel(
        body,
        out_type=jax.ShapeDtypeStruct((NW, ECAP, 8), jnp.float32),
        mesh=_mesh(),
        compiler_params=pltpu.CompilerParams(use_tc_tiling_on_sc=False, needs_layout_passes=False),
        scratch_types=[
            pltpu.VMEM((128,), jnp.int32),
            pltpu.VMEM((128,), jnp.int32),
            pltpu.VMEM((128,), jnp.int32),
            pltpu.VMEM((128, 16), jnp.float32),
            pltpu.VMEM((128, 16), jnp.float32),
            pltpu.VMEM((128, 8), jnp.float32),
            pltpu.VMEM((16,), jnp.int32),
        ],
    )
    return f(p, rz, bdst, bk, counts)


def _phase2(hflat, ab, bsrc, bdst, counts, t, n, C):
    def body(h_h, ab_h, bsrc_h, bdst_h, cnt_h, t_h, num_h, den_h,
             srcv, srcv2, hrows, nacc, dacc, dst_vm, al_vm, tv, csm):
        w = _wid()
        pltpu.sync_copy(cnt_h.at[w, :], csm)
        pltpu.sync_copy(t_h, tv)
        nb = jnp.max(csm[...]) // 128
        tvec = tv[...]
        NC16 = C // 16

        def chunk_body(head, _):
            def zb(r, _2):
                for c in range(NC16):
                    nacc[r, pl.ds(c * 16, 16)] = jnp.zeros(16, jnp.float32)
                    dacc[r, pl.ds(c * 16, 16)] = jnp.zeros(16, jnp.float32)
                return 0
            lax.fori_loop(0, NPT + 1, zb, 0)

            def bat_body(jb, _2):
                off = pl.multiple_of(jb * 128, 128)
                pltpu.sync_copy(bsrc_h.at[w, pl.ds(off, 128)], srcv)
                pltpu.sync_copy(bdst_h.at[w, pl.ds(off, 128)], dst_vm)
                pltpu.sync_copy(ab_h.at[w, pl.ds(off, 128), :], al_vm)
                for i in range(8):
                    srcv2[pl.ds(i * 16, 16)] = (srcv[pl.ds(i * 16, 16)]
                                                + head * n)
                pltpu.sync_copy(h_h.at[srcv2], hrows)
                iota = _iota16()
                fh = jnp.full(16, 0, jnp.int32) + head

                def grp_body(gi, _3):
                    gb = gi * 16
                    d16 = dst_vm[pl.ds(gb, 16)]
                    av16 = plsc.load_gather(al_vm, [gb + iota, fh])
                    for le in range(16):
                        sel = jnp.full((16,), le, jnp.int32)
                        al = _dyn_gather(av16, sel)
                        dspl = _dyn_gather(d16, sel)
                        for c in range(NC16):
                            hv = hrows[gb + le, pl.ds(c * 16, 16)]
                            u = hv * al
                            m = u * tvec
                            g = jnp.exp(m)
                            q = g * u
                            colv = c * 16 + iota
                            plsc.addupdate_scatter(dacc, [dspl, colv], g)
                            plsc.addupdate_scatter(nacc, [dspl, colv], q)
                    return 0

                lax.fori_loop(0, 8, grp_body, 0)
                return 0

            lax.fori_loop(0, nb, bat_body, 0)
            hc = pl.multiple_of(head * C, C)
            rw = pl.multiple_of(w * NPT, NPT)
            pltpu.sync_copy(nacc.at[pl.ds(0, NPT), :],
                            num_h.at[pl.ds(rw, NPT), pl.ds(hc, C)])
            pltpu.sync_copy(dacc.at[pl.ds(0, NPT), :],
                            den_h.at[pl.ds(rw, NPT), pl.ds(hc, C)])
            return 0

        lax.fori_loop(0, 8, chunk_body, 0)

    f = pl.kernel(
        body,
        out_type=(
            jax.ShapeDtypeStruct((NROW, 8 * C), jnp.float32),
            jax.ShapeDtypeStruct((NROW, 8 * C), jnp.float32),
        ),
        mesh=_mesh(),
        compiler_params=pltpu.CompilerParams(use_tc_tiling_on_sc=False, needs_layout_passes=False),
        scratch_types=[
            pltpu.VMEM((128,), jnp.int32),
            pltpu.VMEM((128,), jnp.int32),
            pltpu.VMEM((128, C), jnp.float32),
            pltpu.VMEM((NPT + 1, C), jnp.float32),
            pltpu.VMEM((NPT + 1, C), jnp.float32),
            pltpu.VMEM((128,), jnp.int32),
            pltpu.VMEM((128, 8), jnp.float32),
            pltpu.VMEM((16,), jnp.float32),
            pltpu.VMEM((16,), jnp.int32),
        ],
    )
    return f(hflat, ab, bsrc, bdst, counts, t)


def _mmh_body(x_ref, wt_ref, o_ref):
    o_ref[0] = lax.dot_general(x_ref[...], wt_ref[0],
                               (((1,), (1,)), ((), ())),
                               preferred_element_type=jnp.float32)


def _matmul_heads(x, wt, C, block_n=2000):
    n, d = x.shape
    grid = (n // block_n, 8)
    return pl.pallas_call(
        _mmh_body,
        grid=grid,
        in_specs=[
            pl.BlockSpec((block_n, d), lambda i, h: (i, 0)),
            pl.BlockSpec((1, C, d), lambda i, h: (h, 0, 0)),
        ],
        out_specs=pl.BlockSpec((1, block_n, C), lambda i, h: (h, i, 0)),
        out_shape=jax.ShapeDtypeStruct((8, n, C), jnp.float32),
    )(x, wt)


def _mm_body(x_ref, w_ref, o_ref):
    o_ref[...] = jnp.dot(x_ref[...], w_ref[...],
                         preferred_element_type=jnp.float32)


def _matmul(x, w, block_n=2000):
    n, d = x.shape
    f = w.shape[1]
    return pl.pallas_call(
        _mm_body,
        grid=(n // block_n,),
        in_specs=[
            pl.BlockSpec((block_n, d), lambda i: (i, 0)),
            pl.BlockSpec((d, f), lambda i: (0, 0)),
        ],
        out_specs=pl.BlockSpec((block_n, f), lambda i: (i, 0)),
        out_shape=jax.ShapeDtypeStruct((n, f), jnp.float32),
    )(x, w)


def _mid_body(z_ref, o_ref):
    o_ref[...] = 1.0 / (z_ref[0] + z_ref[1] + 1e-16)


def _mid(zp):
    z2 = zp.reshape(NSC, NPZ // 16, 256)
    rz = pl.pallas_call(
        _mid_body,
        in_specs=[pl.BlockSpec((NSC, NPZ // 16, 256), lambda: (0, 0, 0))],
        out_specs=pl.BlockSpec((NPZ // 16, 256), lambda: (0, 0)),
        out_shape=jax.ShapeDtypeStruct((NPZ // 16, 256), jnp.float32),
    )(z2)
    return rz.reshape(NPZ, 16)


def _fin_body(num_ref, den_ref, b_ref, o_ref):
    o_ref[...] = jax.nn.relu(num_ref[...] / (den_ref[...] + 1e-16)
                             + b_ref[...])


def _finalize(num, den, b, n, C, block_n=2000):
    F = 8 * C
    return pl.pallas_call(
        _fin_body,
        grid=(n // block_n,),
        in_specs=[
            pl.BlockSpec((block_n, F), lambda i: (i, 0)),
            pl.BlockSpec((block_n, F), lambda i: (i, 0)),
            pl.BlockSpec((1, F), lambda i: (0, 0)),
        ],
        out_specs=pl.BlockSpec((block_n, F), lambda i: (i, 0)),
        out_shape=jax.ShapeDtypeStruct((n, F), jnp.float32),
    )(num, den, b.reshape(1, F))


def _gat_layer(x, srcr, dstr, bsrc, bdst, bk, counts, W, att_src, att_dst,
               b, t, E):
    n = x.shape[0]
    heads, ch = att_src.shape
    d = W.shape[0]
    ws = (W.reshape(d, heads, ch) * att_src[None]).sum(-1)
    wd = (W.reshape(d, heads, ch) * att_dst[None]).sum(-1)
    wt = W.T.reshape(heads, ch, d)
    h_hm = _matmul_heads(x, wt, ch)
    a_cat = _matmul(x, jnp.concatenate([ws, wd], axis=1))
    p, zp = _phase1(a_cat, srcr, dstr, E)
    rz = _mid(zp)
    ab = _phase1c(p, rz, bdst, bk, counts, E)
    num, den = _phase2(h_hm.reshape(heads * n, ch), ab, bsrc, bdst, counts,
                       jnp.full((16,), t, jnp.float32), n, ch)
    return _finalize(num, den, b, n, ch)


def kernel(x, edge_index, W1, as1, ad1, b1, t1, W2, as2, ad2, b2, t2):
    E = edge_index.shape[1]
    srcr = edge_index[0].reshape(E // 128, 128)
    dstr = edge_index[1].reshape(E // 128, 128)
    bsrc, bdst, bk, counts = _phase0(srcr, dstr, E)
    h1 = _gat_layer(x, srcr, dstr, bsrc, bdst, bk, counts,
                    W1, as1, ad1, b1, t1, E)
    return _gat_layer(h1, srcr, dstr, bsrc, bdst, bk, counts,
                      W2, as2, ad2, b2, t2, E)


# speedup vs baseline: 2.2003x; 2.2003x over previous
import functools

import jax
import jax.numpy as jnp
from jax import lax
from jax.experimental import pallas as pl
from jax.experimental.pallas import tpu as pltpu
from jax.experimental.pallas import tpu_sc as plsc

NSC, NSUB = 2, 16
NW = NSC * NSUB
NPT = 320
NROW = NW * NPT
NPZ = 10368
ZSTRIPE = NPZ // NSUB


def _mesh():
    return plsc.VectorSubcoreMesh(core_axis_name="c", subcore_axis_name="s")


def _wid():
    return lax.axis_index("s") * NSC + lax.axis_index("c")


def _iota16():
    return lax.iota(jnp.int32, 16)


def _dyn_gather(x, idx):
    return x.at[idx].get(mode="promise_in_bounds")


def _phase0(srcr, dstr, E):
    nrows = E // 128
    SR = 20
    nchunks = nrows // SR
    ECAP = E + 128

    def body(srcr_h, dstr_h, bsrc_h, bdst_h, bk_h, cnt_h,
             sv, dv, st_s, st_d, st_k, csm):
        w = _wid()
        lo = w * NPT
        iota = _iota16()

        def chunk_body(ci, carry):
            cur, out = carry
            rbase = ci * SR
            pltpu.sync_copy(srcr_h.at[pl.ds(rbase, SR), :], sv)
            pltpu.sync_copy(dstr_h.at[pl.ds(rbase, SR), :], dv)

            def group_body(g, carry2):
                cur2, = carry2
                j = g // 8
                sub = g % 8
                d16 = dv[j, pl.ds(sub * 16, 16)]
                s16 = sv[j, pl.ds(sub * 16, 16)]
                k16 = (ci * (SR * 128) + g * 16) + iota
                msk = (d16 >= lo) & (d16 < lo + NPT)
                plsc.store_compressed(st_s.at[pl.ds(cur2, 16)], s16, mask=msk)
                plsc.store_compressed(st_d.at[pl.ds(cur2, 16)], d16 - lo,
                                      mask=msk)
                plsc.store_compressed(st_k.at[pl.ds(cur2, 16)], k16, mask=msk)
                cnt = jnp.sum(msk.astype(jnp.int32))
                return (cur2 + cnt,)

            cur, = lax.fori_loop(0, SR * 8, group_body, (cur,))

            nfl = cur // 128

            def flush_body(j, _):
                o = pl.multiple_of(out + j * 128, 128)
                pltpu.sync_copy(st_s.at[pl.ds(j * 128, 128)],
                                bsrc_h.at[w, pl.ds(o, 128)])
                pltpu.sync_copy(st_d.at[pl.ds(j * 128, 128)],
                                bdst_h.at[w, pl.ds(o, 128)])
                pltpu.sync_copy(st_k.at[pl.ds(j * 128, 128)],
                                bk_h.at[w, pl.ds(o, 128)])
                return 0

            lax.fori_loop(0, nfl, flush_body, 0)
            tb = nfl * 128
            for i in range(8):
                st_s[pl.ds(i * 16, 16)] = st_s[pl.ds(tb + i * 16, 16)]
                st_d[pl.ds(i * 16, 16)] = st_d[pl.ds(tb + i * 16, 16)]
                st_k[pl.ds(i * 16, 16)] = st_k[pl.ds(tb + i * 16, 16)]
            return (cur - tb, out + tb)

        cur, out = lax.fori_loop(0, nchunks, chunk_body, (0, 0))

        @pl.when(cur > 0)
        def _():
            for i in range(8):
                st_s[pl.ds(cur + i * 16, 16)] = jnp.zeros(16, jnp.int32)
                st_d[pl.ds(cur + i * 16, 16)] = jnp.full(16, NPT, jnp.int32)
                st_k[pl.ds(cur + i * 16, 16)] = jnp.zeros(16, jnp.int32)
            o = pl.multiple_of(out, 128)
            pltpu.sync_copy(st_s.at[pl.ds(0, 128)],
                            bsrc_h.at[w, pl.ds(o, 128)])
            pltpu.sync_copy(st_d.at[pl.ds(0, 128)],
                            bdst_h.at[w, pl.ds(o, 128)])
            pltpu.sync_copy(st_k.at[pl.ds(0, 128)],
                            bk_h.at[w, pl.ds(o, 128)])

        total = jnp.where(cur > 0, out + 128, out)
        csm[:] = jnp.full(16, total, jnp.int32)
        pltpu.sync_copy(csm, cnt_h.at[w, :])

    f = pl.kernel(
        body,
        out_type=(
            jax.ShapeDtypeStruct((NW, ECAP), jnp.int32),
            jax.ShapeDtypeStruct((NW, ECAP), jnp.int32),
            jax.ShapeDtypeStruct((NW, ECAP), jnp.int32),
            jax.ShapeDtypeStruct((NW, 16), jnp.int32),
        ),
        mesh=_mesh(),
        compiler_params=pltpu.CompilerParams(use_tc_tiling_on_sc=False, needs_layout_passes=False),
        scratch_types=[
            pltpu.VMEM((SR, 128), jnp.int32),
            pltpu.VMEM((SR, 128), jnp.int32),
            pltpu.VMEM((2704,), jnp.int32),
            pltpu.VMEM((2704,), jnp.int32),
            pltpu.VMEM((2704,), jnp.int32),
            pltpu.VMEM((16,), jnp.int32),
        ],
    )
    return f(srcr, dstr)


def _phase1(a_cat, srcr, dstr, E):
    NBLK = E // 512
    npw = (NBLK + NW - 1) // NW

    def body(a_h, srcr_h, dstr_h, p_h, zp_h,
             sv, dv, as_r, ad_r, pb, zb, z_sh):
        w = _wid()
        cid = lax.axis_index("c")
        sid = lax.axis_index("s")
        iota = _iota16()

        def zp_body(r, _):
            pb[r, :] = jnp.zeros(16, jnp.float32)
            return 0
        lax.fori_loop(0, 512, zp_body, 0)

        def zz_body(r, _):
            zb[r, :] = jnp.zeros(16, jnp.float32)
            return 0
        lax.fori_loop(0, ZSTRIPE, zz_body, 0)
        pltpu.sync_copy(zb, z_sh.at[pl.ds(pl.multiple_of(sid * ZSTRIPE, 8),
                                          ZSTRIPE), :])
        plsc.subcore_barrier()

        def blk_body(j, _):
            b = j * NW + w

            @pl.when(b < NBLK)
            def _():
                rbase = b * 4
                pltpu.sync_copy(srcr_h.at[pl.ds(rbase, 4), :], sv)
                pltpu.sync_copy(dstr_h.at[pl.ds(rbase, 4), :], dv)
                for jj in range(4):
                    pltpu.sync_copy(a_h.at[sv.at[jj]],
                                    as_r.at[pl.ds(jj * 128, 128), :])
                    pltpu.sync_copy(a_h.at[dv.at[jj]],
                                    ad_r.at[pl.ds(jj * 128, 128), :])

                def grp_body(g, _2):
                    e16 = g * 16 + iota
                    for h in range(8):
                        fh = jnp.full(16, h, jnp.int32)
                        av = plsc.load_gather(as_r, [e16, fh])
                        bv = plsc.load_gather(ad_r, [e16, fh + 8])
                        ev = av + bv
                        lv = jnp.maximum(ev, 0.0) + 0.2 * jnp.minimum(ev, 0.0)
                        pv = jnp.exp(lv)
                        plsc.store_scatter(pb, [e16, fh], pv)
                    return 0

                lax.fori_loop(0, 32, grp_body, 0)
                pltpu.sync_copy(pb, p_h.at[pl.ds(pl.multiple_of(b * 512, 512), 512), :])
                for jj in range(4):
                    pltpu.sync_copy(pb.at[pl.ds(jj * 128, 128), :],
                                    z_sh.at[dv.at[jj]], add=True)
            return 0

        lax.fori_loop(0, npw, blk_body, 0)
        plsc.subcore_barrier()
        st8 = pl.multiple_of(sid * ZSTRIPE, 8)
        pltpu.sync_copy(z_sh.at[pl.ds(st8, ZSTRIPE), :],
                        zp_h.at[cid, pl.ds(st8, ZSTRIPE), :])

    f = pl.kernel(
        body,
        out_type=(
            jax.ShapeDtypeStruct((E, 16), jnp.float32),
            jax.ShapeDtypeStruct((NSC, NPZ, 16), jnp.float32),
        ),
        mesh=_mesh(),
        compiler_params=pltpu.CompilerParams(use_tc_tiling_on_sc=False, needs_layout_passes=False),
        scratch_types=[
            pltpu.VMEM((4, 128), jnp.int32),
            pltpu.VMEM((4, 128), jnp.int32),
            pltpu.VMEM((512, 16), jnp.float32),
            pltpu.VMEM((512, 16), jnp.float32),
            pltpu.VMEM((512, 16), jnp.float32),
            pltpu.VMEM((ZSTRIPE, 16), jnp.float32),
            pltpu.VMEM_SHARED((NPZ, 16), jnp.float32),
        ],
    )
    return f(a_cat, srcr, dstr)


def _phase1c(p, rz, bdst, bk, counts, E):
    ECAP = E + 128

    def body(p_h, rz_h, bdst_h, bk_h, cnt_h, ab_h,
             kb, dlb, dgb, prows, zrows, abuf, csm):
        w = _wid()
        iota = _iota16()
        pltpu.sync_copy(cnt_h.at[w, :], csm)
        nb = jnp.max(csm[...]) // 128

        def bat_body(jb, _):
            off = pl.multiple_of(jb * 128, 128)
            pltpu.sync_copy(bk_h.at[w, pl.ds(off, 128)], kb)
            pltpu.sync_copy(bdst_h.at[w, pl.ds(off, 128)], dlb)
            for i in range(8):
                dgb[pl.ds(i * 16, 16)] = dlb[pl.ds(i * 16, 16)] + w * NPT
            pltpu.sync_copy(p_h.at[kb], prows)
            pltpu.sync_copy(rz_h.at[dgb], zrows)
            for i in range(8):
                e16 = i * 16 + iota
                for h in range(8):
                    fh = jnp.full(16, h, jnp.int32)
                    av = (plsc.load_gather(prows, [e16, fh])
                          * plsc.load_gather(zrows, [e16, fh]))
                    plsc.store_scatter(abuf, [e16, fh], av)
            pltpu.sync_copy(abuf, ab_h.at[w, pl.ds(off, 128), :])
            return 0

        lax.fori_loop(0, nb, bat_body, 0)

    f = pl.kernel(
        body,
        out_type=jax.ShapeDtypeStruct((NW, ECAP, 8), jnp.float32),
        mesh=_mesh(),
        compiler_params=pltpu.CompilerParams(use_tc_tiling_on_sc=False, needs_layout_passes=False),
        scratch_types=[
            pltpu.VMEM((128,), jnp.int32),
            pltpu.VMEM((128,), jnp.int32),
            pltpu.VMEM((128,), jnp.int32),
            pltpu.VMEM((128, 16), jnp.float32),
            pltpu.VMEM((128, 16), jnp.float32),
            pltpu.VMEM((128, 8), jnp.float32),
            pltpu.VMEM((16,), jnp.int32),
        ],
    )
    return f(p, rz, bdst, bk, counts)


def _phase2(hflat, ab, bsrc, bdst, counts, t, n, C):
    def body(h_h, ab_h, bsrc_h, bdst_h, cnt_h, t_h, num_h, den_h,
             srcv, srcv2, hrows, nacc, dacc, dst_vm, al_vm, tv, csm):
        w = _wid()
        pltpu.sync_copy(cnt_h.at[w, :], csm)
        pltpu.sync_copy(t_h, tv)
        nb = jnp.max(csm[...]) // 128
        tvec = tv[...]
        NC16 = C // 16

        def chunk_body(head, _):
            def zb(r, _2):
                for c in range(NC16):
                    nacc[r, pl.ds(c * 16, 16)] = jnp.zeros(16, jnp.float32)
                    dacc[r, pl.ds(c * 16, 16)] = jnp.zeros(16, jnp.float32)
                return 0
            lax.fori_loop(0, NPT + 1, zb, 0)

            def bat_body(jb, _2):
                off = pl.multiple_of(jb * 128, 128)
                pltpu.sync_copy(bsrc_h.at[w, pl.ds(off, 128)], srcv)
                pltpu.sync_copy(bdst_h.at[w, pl.ds(off, 128)], dst_vm)
                pltpu.sync_copy(ab_h.at[w, pl.ds(off, 128), :], al_vm)
                for i in range(8):
                    srcv2[pl.ds(i * 16, 16)] = (srcv[pl.ds(i * 16, 16)]
                                                + head * n)
                pltpu.sync_copy(h_h.at[srcv2], hrows)
                iota = _iota16()
                fh = jnp.full(16, 0, jnp.int32) + head

                def grp_body(gi, _3):
                    gb = gi * 16
                    d16 = dst_vm[pl.ds(gb, 16)]
                    av16 = plsc.load_gather(al_vm, [gb + iota, fh])
                    for le in range(16):
                        sel = jnp.full((16,), le, jnp.int32)
                        al = _dyn_gather(av16, sel)
                        dspl = _dyn_gather(d16, sel)
                        hvs = [hrows[gb + le, pl.ds(c * 16, 16)]
                               for c in range(NC16)]
                        us = [hv * al for hv in hvs]
                        gs = [jnp.exp(u * tvec) for u in us]
                        qs = [g * u for g, u in zip(gs, us)]
                        for c in range(NC16):
                            colv = c * 16 + iota
                            plsc.addupdate_scatter(dacc, [dspl, colv], gs[c])
                            plsc.addupdate_scatter(nacc, [dspl, colv], qs[c])
                    return 0

                lax.fori_loop(0, 8, grp_body, 0)
                return 0

            lax.fori_loop(0, nb, bat_body, 0)
            hc = pl.multiple_of(head * C, C)
            rw = pl.multiple_of(w * NPT, NPT)
            pltpu.sync_copy(nacc.at[pl.ds(0, NPT), :],
                            num_h.at[pl.ds(rw, NPT), pl.ds(hc, C)])
            pltpu.sync_copy(dacc.at[pl.ds(0, NPT), :],
                            den_h.at[pl.ds(rw, NPT), pl.ds(hc, C)])
            return 0

        lax.fori_loop(0, 8, chunk_body, 0)

    f = pl.kernel(
        body,
        out_type=(
            jax.ShapeDtypeStruct((NROW, 8 * C), jnp.float32),
            jax.ShapeDtypeStruct((NROW, 8 * C), jnp.float32),
        ),
        mesh=_mesh(),
        compiler_params=pltpu.CompilerParams(use_tc_tiling_on_sc=False, needs_layout_passes=False),
        scratch_types=[
            pltpu.VMEM((128,), jnp.int32),
            pltpu.VMEM((128,), jnp.int32),
            pltpu.VMEM((128, C), jnp.float32),
            pltpu.VMEM((NPT + 1, C), jnp.float32),
            pltpu.VMEM((NPT + 1, C), jnp.float32),
            pltpu.VMEM((128,), jnp.int32),
            pltpu.VMEM((128, 8), jnp.float32),
            pltpu.VMEM((16,), jnp.float32),
            pltpu.VMEM((16,), jnp.int32),
        ],
    )
    return f(hflat, ab, bsrc, bdst, counts, t)


def _mmh_body(x_ref, wt_ref, o_ref):
    o_ref[0] = lax.dot_general(x_ref[...], wt_ref[0],
                               (((1,), (1,)), ((), ())),
                               preferred_element_type=jnp.float32)


def _matmul_heads(x, wt, C, block_n=2000):
    n, d = x.shape
    grid = (n // block_n, 8)
    return pl.pallas_call(
        _mmh_body,
        grid=grid,
        in_specs=[
            pl.BlockSpec((block_n, d), lambda i, h: (i, 0)),
            pl.BlockSpec((1, C, d), lambda i, h: (h, 0, 0)),
        ],
        out_specs=pl.BlockSpec((1, block_n, C), lambda i, h: (h, i, 0)),
        out_shape=jax.ShapeDtypeStruct((8, n, C), jnp.float32),
    )(x, wt)


def _mm_body(x_ref, w_ref, o_ref):
    o_ref[...] = jnp.dot(x_ref[...], w_ref[...],
                         preferred_element_type=jnp.float32)


def _matmul(x, w, block_n=2000):
    n, d = x.shape
    f = w.shape[1]
    return pl.pallas_call(
        _mm_body,
        grid=(n // block_n,),
        in_specs=[
            pl.BlockSpec((block_n, d), lambda i: (i, 0)),
            pl.BlockSpec((d, f), lambda i: (0, 0)),
        ],
        out_specs=pl.BlockSpec((block_n, f), lambda i: (i, 0)),
        out_shape=jax.ShapeDtypeStruct((n, f), jnp.float32),
    )(x, w)


def _mid_body(z_ref, o_ref):
    o_ref[...] = 1.0 / (z_ref[0] + z_ref[1] + 1e-16)


def _mid(zp):
    z2 = zp.reshape(NSC, NPZ // 16, 256)
    rz = pl.pallas_call(
        _mid_body,
        in_specs=[pl.BlockSpec((NSC, NPZ // 16, 256), lambda: (0, 0, 0))],
        out_specs=pl.BlockSpec((NPZ // 16, 256), lambda: (0, 0)),
        out_shape=jax.ShapeDtypeStruct((NPZ // 16, 256), jnp.float32),
    )(z2)
    return rz.reshape(NPZ, 16)


def _fin_body(num_ref, den_ref, b_ref, o_ref):
    o_ref[...] = jax.nn.relu(num_ref[...] / (den_ref[...] + 1e-16)
                             + b_ref[...])


def _finalize(num, den, b, n, C, block_n=2000):
    F = 8 * C
    return pl.pallas_call(
        _fin_body,
        grid=(n // block_n,),
        in_specs=[
            pl.BlockSpec((block_n, F), lambda i: (i, 0)),
            pl.BlockSpec((block_n, F), lambda i: (i, 0)),
            pl.BlockSpec((1, F), lambda i: (0, 0)),
        ],
        out_specs=pl.BlockSpec((block_n, F), lambda i: (i, 0)),
        out_shape=jax.ShapeDtypeStruct((n, F), jnp.float32),
    )(num, den, b.reshape(1, F))


def _gat_layer(x, srcr, dstr, bsrc, bdst, bk, counts, W, att_src, att_dst,
               b, t, E):
    n = x.shape[0]
    heads, ch = att_src.shape
    d = W.shape[0]
    ws = (W.reshape(d, heads, ch) * att_src[None]).sum(-1)
    wd = (W.reshape(d, heads, ch) * att_dst[None]).sum(-1)
    wt = W.T.reshape(heads, ch, d)
    h_hm = _matmul_heads(x, wt, ch)
    a_cat = _matmul(x, jnp.concatenate([ws, wd], axis=1))
    p, zp = _phase1(a_cat, srcr, dstr, E)
    rz = _mid(zp)
    ab = _phase1c(p, rz, bdst, bk, counts, E)
    num, den = _phase2(h_hm.reshape(heads * n, ch), ab, bsrc, bdst, counts,
                       jnp.full((16,), t, jnp.float32), n, ch)
    return _finalize(num, den, b, n, ch)


def kernel(x, edge_index, W1, as1, ad1, b1, t1, W2, as2, ad2, b2, t2):
    E = edge_index.shape[1]
    srcr = edge_index[0].reshape(E // 128, 128)
    dstr = edge_index[1].reshape(E // 128, 128)
    bsrc, bdst, bk, counts = _phase0(srcr, dstr, E)
    h1 = _gat_layer(x, srcr, dstr, bsrc, bdst, bk, counts,
                    W1, as1, ad1, b1, t1, E)
    return _gat_layer(h1, srcr, dstr, bsrc, bdst, bk, counts,
                      W2, as2, ad2, b2, t2, E)


# speedup vs baseline: 3.0893x; 1.4040x over previous
import functools

import jax
import jax.numpy as jnp
from jax import lax
from jax.experimental import pallas as pl
from jax.experimental.pallas import tpu as pltpu
from jax.experimental.pallas import tpu_sc as plsc

NSC, NSUB = 2, 16
NW = NSC * NSUB
NPT = 320
NROW = NW * NPT
NPZ = 10368
ZSTRIPE = NPZ // NSUB


def _mesh():
    return plsc.VectorSubcoreMesh(core_axis_name="c", subcore_axis_name="s")


def _wid():
    return lax.axis_index("s") * NSC + lax.axis_index("c")


def _iota16():
    return lax.iota(jnp.int32, 16)


def _dyn_gather(x, idx):
    return x.at[idx].get(mode="promise_in_bounds")


def _phase0(srcr, dstr, E):
    nrows = E // 128
    SR = 20
    nchunks = nrows // SR
    ECAP = E + 128

    def body(srcr_h, dstr_h, bsrc_h, bdst_h, bk_h, cnt_h,
             sv, dv, st_s, st_d, st_k, csm):
        w = _wid()
        lo = w * NPT
        iota = _iota16()

        def chunk_body(ci, carry):
            cur, out = carry
            rbase = ci * SR
            pltpu.sync_copy(srcr_h.at[pl.ds(rbase, SR), :], sv)
            pltpu.sync_copy(dstr_h.at[pl.ds(rbase, SR), :], dv)

            def group_body(g, carry2):
                cur2, = carry2
                j = g // 8
                sub = g % 8
                d16 = dv[j, pl.ds(sub * 16, 16)]
                s16 = sv[j, pl.ds(sub * 16, 16)]
                k16 = (ci * (SR * 128) + g * 16) + iota
                msk = (d16 >= lo) & (d16 < lo + NPT)
                plsc.store_compressed(st_s.at[pl.ds(cur2, 16)], s16, mask=msk)
                plsc.store_compressed(st_d.at[pl.ds(cur2, 16)], d16 - lo,
                                      mask=msk)
                plsc.store_compressed(st_k.at[pl.ds(cur2, 16)], k16, mask=msk)
                cnt = jnp.sum(msk.astype(jnp.int32))
                return (cur2 + cnt,)

            cur, = lax.fori_loop(0, SR * 8, group_body, (cur,))

            nfl = cur // 128

            def flush_body(j, _):
                o = pl.multiple_of(out + j * 128, 128)
                pltpu.sync_copy(st_s.at[pl.ds(j * 128, 128)],
                                bsrc_h.at[w, pl.ds(o, 128)])
                pltpu.sync_copy(st_d.at[pl.ds(j * 128, 128)],
                                bdst_h.at[w, pl.ds(o, 128)])
                pltpu.sync_copy(st_k.at[pl.ds(j * 128, 128)],
                                bk_h.at[w, pl.ds(o, 128)])
                return 0

            lax.fori_loop(0, nfl, flush_body, 0)
            tb = nfl * 128
            for i in range(8):
                st_s[pl.ds(i * 16, 16)] = st_s[pl.ds(tb + i * 16, 16)]
                st_d[pl.ds(i * 16, 16)] = st_d[pl.ds(tb + i * 16, 16)]
                st_k[pl.ds(i * 16, 16)] = st_k[pl.ds(tb + i * 16, 16)]
            return (cur - tb, out + tb)

        cur, out = lax.fori_loop(0, nchunks, chunk_body, (0, 0))

        @pl.when(cur > 0)
        def _():
            for i in range(8):
                st_s[pl.ds(cur + i * 16, 16)] = jnp.zeros(16, jnp.int32)
                st_d[pl.ds(cur + i * 16, 16)] = jnp.full(16, NPT, jnp.int32)
                st_k[pl.ds(cur + i * 16, 16)] = jnp.zeros(16, jnp.int32)
            o = pl.multiple_of(out, 128)
            pltpu.sync_copy(st_s.at[pl.ds(0, 128)],
                            bsrc_h.at[w, pl.ds(o, 128)])
            pltpu.sync_copy(st_d.at[pl.ds(0, 128)],
                            bdst_h.at[w, pl.ds(o, 128)])
            pltpu.sync_copy(st_k.at[pl.ds(0, 128)],
                            bk_h.at[w, pl.ds(o, 128)])

        total = jnp.where(cur > 0, out + 128, out)
        csm[:] = jnp.full(16, total, jnp.int32)
        pltpu.sync_copy(csm, cnt_h.at[w, :])

    f = pl.kernel(
        body,
        out_type=(
            jax.ShapeDtypeStruct((NW, ECAP), jnp.int32),
            jax.ShapeDtypeStruct((NW, ECAP), jnp.int32),
            jax.ShapeDtypeStruct((NW, ECAP), jnp.int32),
            jax.ShapeDtypeStruct((NW, 16), jnp.int32),
        ),
        mesh=_mesh(),
        compiler_params=pltpu.CompilerParams(use_tc_tiling_on_sc=False, needs_layout_passes=False),
        scratch_types=[
            pltpu.VMEM((SR, 128), jnp.int32),
            pltpu.VMEM((SR, 128), jnp.int32),
            pltpu.VMEM((2704,), jnp.int32),
            pltpu.VMEM((2704,), jnp.int32),
            pltpu.VMEM((2704,), jnp.int32),
            pltpu.VMEM((16,), jnp.int32),
        ],
    )
    return f(srcr, dstr)


def _phase1(a_cat, srcr, dstr, E):
    NBLK = E // 512
    npw = (NBLK + NW - 1) // NW

    def body(a_h, srcr_h, dstr_h, p_h, zp_h,
             sv, dv, as_r, ad_r, pb, zb, z_sh):
        w = _wid()
        cid = lax.axis_index("c")
        sid = lax.axis_index("s")
        iota = _iota16()

        def zp_body(r, _):
            pb[r, :] = jnp.zeros(16, jnp.float32)
            return 0
        lax.fori_loop(0, 512, zp_body, 0)

        def zz_body(r, _):
            zb[r, :] = jnp.zeros(16, jnp.float32)
            return 0
        lax.fori_loop(0, ZSTRIPE, zz_body, 0)
        pltpu.sync_copy(zb, z_sh.at[pl.ds(pl.multiple_of(sid * ZSTRIPE, 8),
                                          ZSTRIPE), :])
        plsc.subcore_barrier()

        def blk_body(j, _):
            b = j * NW + w

            @pl.when(b < NBLK)
            def _():
                rbase = b * 4
                pltpu.sync_copy(srcr_h.at[pl.ds(rbase, 4), :], sv)
                pltpu.sync_copy(dstr_h.at[pl.ds(rbase, 4), :], dv)
                for jj in range(4):
                    pltpu.sync_copy(a_h.at[sv.at[jj]],
                                    as_r.at[pl.ds(jj * 128, 128), :])
                    pltpu.sync_copy(a_h.at[dv.at[jj]],
                                    ad_r.at[pl.ds(jj * 128, 128), :])

                def grp_body(g, _2):
                    e16 = g * 16 + iota
                    for h in range(8):
                        fh = jnp.full(16, h, jnp.int32)
                        av = plsc.load_gather(as_r, [e16, fh])
                        bv = plsc.load_gather(ad_r, [e16, fh + 8])
                        ev = av + bv
                        lv = jnp.maximum(ev, 0.0) + 0.2 * jnp.minimum(ev, 0.0)
                        pv = jnp.exp(lv)
                        plsc.store_scatter(pb, [e16, fh], pv)
                    return 0

                lax.fori_loop(0, 32, grp_body, 0)
                pltpu.sync_copy(pb, p_h.at[pl.ds(pl.multiple_of(b * 512, 512), 512), :])
                for jj in range(4):
                    pltpu.sync_copy(pb.at[pl.ds(jj * 128, 128), :],
                                    z_sh.at[dv.at[jj]], add=True)
            return 0

        lax.fori_loop(0, npw, blk_body, 0)
        plsc.subcore_barrier()
        st8 = pl.multiple_of(sid * ZSTRIPE, 8)
        pltpu.sync_copy(z_sh.at[pl.ds(st8, ZSTRIPE), :],
                        zp_h.at[cid, pl.ds(st8, ZSTRIPE), :])

    f = pl.kernel(
        body,
        out_type=(
            jax.ShapeDtypeStruct((E, 16), jnp.float32),
            jax.ShapeDtypeStruct((NSC, NPZ, 16), jnp.float32),
        ),
        mesh=_mesh(),
        compiler_params=pltpu.CompilerParams(use_tc_tiling_on_sc=False, needs_layout_passes=False),
        scratch_types=[
            pltpu.VMEM((4, 128), jnp.int32),
            pltpu.VMEM((4, 128), jnp.int32),
            pltpu.VMEM((512, 16), jnp.float32),
            pltpu.VMEM((512, 16), jnp.float32),
            pltpu.VMEM((512, 16), jnp.float32),
            pltpu.VMEM((ZSTRIPE, 16), jnp.float32),
            pltpu.VMEM_SHARED((NPZ, 16), jnp.float32),
        ],
    )
    return f(a_cat, srcr, dstr)


def _phase1c(p, rz, bdst, bk, counts, E):
    ECAP = E + 128

    def body(p_h, rz_h, bdst_h, bk_h, cnt_h, ab_h,
             kb, dlb, dgb, prows, zrows, abuf, csm):
        w = _wid()
        iota = _iota16()
        pltpu.sync_copy(cnt_h.at[w, :], csm)
        nb = jnp.max(csm[...]) // 128

        def bat_body(jb, _):
            off = pl.multiple_of(jb * 128, 128)
            pltpu.sync_copy(bk_h.at[w, pl.ds(off, 128)], kb)
            pltpu.sync_copy(bdst_h.at[w, pl.ds(off, 128)], dlb)
            for i in range(8):
                dgb[pl.ds(i * 16, 16)] = dlb[pl.ds(i * 16, 16)] + w * NPT
            pltpu.sync_copy(p_h.at[kb], prows)
            pltpu.sync_copy(rz_h.at[dgb], zrows)
            for i in range(8):
                e16 = i * 16 + iota
                for h in range(8):
                    fh = jnp.full(16, h, jnp.int32)
                    av = (plsc.load_gather(prows, [e16, fh])
                          * plsc.load_gather(zrows, [e16, fh]))
                    plsc.store_scatter(abuf, [e16, fh], av)
            pltpu.sync_copy(abuf, ab_h.at[w, pl.ds(off, 128), :])
            return 0

        lax.fori_loop(0, nb, bat_body, 0)

    f = pl.kernel(
        body,
        out_type=jax.ShapeDtypeStruct((NW, ECAP, 8), jnp.float32),
        mesh=_mesh(),
        compiler_params=pltpu.CompilerParams(use_tc_tiling_on_sc=False, needs_layout_passes=False),
        scratch_types=[
            pltpu.VMEM((128,), jnp.int32),
            pltpu.VMEM((128,), jnp.int32),
            pltpu.VMEM((128,), jnp.int32),
            pltpu.VMEM((128, 16), jnp.float32),
            pltpu.VMEM((128, 16), jnp.float32),
            pltpu.VMEM((128, 8), jnp.float32),
            pltpu.VMEM((16,), jnp.int32),
        ],
    )
    return f(p, rz, bdst, bk, counts)


def _phase2(hflat, ab, bsrc, bdst, counts, t, n, C):
    def body(h_h, ab_h, bsrc_h, bdst_h, cnt_h, t_h, num_h, den_h,
             srcv, srcv2, hrows, nacc, dacc, dst_vm, al_vm, tv, csm,
             sema0, sema1, semh0, semh1):
        w = _wid()
        pltpu.sync_copy(cnt_h.at[w, :], csm)
        pltpu.sync_copy(t_h, tv)
        nb = jnp.max(csm[...]) // 128
        tvec = tv[...]
        NC16 = C // 16
        sema = [sema0, sema1]
        semh = [semh0, semh1]

        def meta_refs(jb, s):
            off = pl.multiple_of(jb * 128, 128)
            return [(bsrc_h.at[w, pl.ds(off, 128)], srcv.at[s]),
                    (bdst_h.at[w, pl.ds(off, 128)], dst_vm.at[s]),
                    (ab_h.at[w, pl.ds(off, 128), :], al_vm.at[s])]

        def issue_meta(jb, s):
            for a, b in meta_refs(jb, s):
                pltpu.async_copy(a, b, sema[s])

        def wait_meta(jb, s):
            for a, b in meta_refs(jb, s):
                pltpu.make_async_copy(a, b, sema[s]).wait()

        def issue_h(s, head):
            for i in range(8):
                srcv2[s, pl.ds(i * 16, 16)] = (srcv[s, pl.ds(i * 16, 16)]
                                               + head * n)
            pltpu.async_copy(h_h.at[srcv2.at[s]], hrows.at[s], semh[s])

        def wait_h(s):
            pltpu.make_async_copy(h_h.at[srcv2.at[s]], hrows.at[s],
                                  semh[s]).wait()

        def chunk_body(head, _):
            def zb(r, _2):
                for c in range(NC16):
                    nacc[r, pl.ds(c * 16, 16)] = jnp.zeros(16, jnp.float32)
                    dacc[r, pl.ds(c * 16, 16)] = jnp.zeros(16, jnp.float32)
                return 0
            lax.fori_loop(0, NPT + 1, zb, 0)
            iota = _iota16()
            fh = jnp.full(16, 0, jnp.int32) + head

            def compute(s):
                def grp_body(gi, _3):
                    gb = gi * 16
                    d16 = dst_vm[s, pl.ds(gb, 16)]
                    av16 = plsc.load_gather(al_vm.at[s], [gb + iota, fh])
                    for le in range(16):
                        sel = jnp.full((16,), le, jnp.int32)
                        al = _dyn_gather(av16, sel)
                        dspl = _dyn_gather(d16, sel)
                        hvs = [hrows[s, gb + le, pl.ds(c * 16, 16)]
                               for c in range(NC16)]
                        us = [hv * al for hv in hvs]
                        gs = [jnp.exp(u * tvec) for u in us]
                        qs = [g * u for g, u in zip(gs, us)]
                        for c in range(NC16):
                            colv = c * 16 + iota
                            plsc.addupdate_scatter(dacc, [dspl, colv], gs[c])
                            plsc.addupdate_scatter(nacc, [dspl, colv], qs[c])
                    return 0

                lax.fori_loop(0, 8, grp_body, 0)

            @pl.when(nb > 0)
            def _():
                issue_meta(0, 0)
                wait_meta(0, 0)
                issue_h(0, head)

            @pl.when(nb > 1)
            def _():
                issue_meta(1, 1)

            def pair_body(jp, _2):
                jb0 = jp * 2
                jb1 = jb0 + 1

                @pl.when(jb1 < nb)
                def _():
                    wait_meta(jb1, 1)
                    issue_h(1, head)
                wait_h(0)
                compute(0)

                @pl.when(jb0 + 2 < nb)
                def _():
                    issue_meta(jb0 + 2, 0)

                @pl.when(jb1 + 1 < nb)
                def _():
                    wait_meta(jb1 + 1, 0)
                    issue_h(0, head)
                wait_h(1)
                compute(1)

                @pl.when(jb1 + 2 < nb)
                def _():
                    issue_meta(jb1 + 2, 1)
                return 0

            lax.fori_loop(0, nb // 2, pair_body, 0)

            @pl.when((nb % 2) == 1)
            def _():
                wait_h(0)
                compute(0)

            hc = pl.multiple_of(head * C, C)
            rw = pl.multiple_of(w * NPT, NPT)
            pltpu.sync_copy(nacc.at[pl.ds(0, NPT), :],
                            num_h.at[pl.ds(rw, NPT), pl.ds(hc, C)])
            pltpu.sync_copy(dacc.at[pl.ds(0, NPT), :],
                            den_h.at[pl.ds(rw, NPT), pl.ds(hc, C)])
            return 0

        lax.fori_loop(0, 8, chunk_body, 0)

    f = pl.kernel(
        body,
        out_type=(
            jax.ShapeDtypeStruct((NROW, 8 * C), jnp.float32),
            jax.ShapeDtypeStruct((NROW, 8 * C), jnp.float32),
        ),
        mesh=_mesh(),
        compiler_params=pltpu.CompilerParams(use_tc_tiling_on_sc=False, needs_layout_passes=False),
        scratch_types=[
            pltpu.VMEM((2, 128), jnp.int32),
            pltpu.VMEM((2, 128), jnp.int32),
            pltpu.VMEM((2, 128, C), jnp.float32),
            pltpu.VMEM((NPT + 1, C), jnp.float32),
            pltpu.VMEM((NPT + 1, C), jnp.float32),
            pltpu.VMEM((2, 128), jnp.int32),
            pltpu.VMEM((2, 128, 8), jnp.float32),
            pltpu.VMEM((16,), jnp.float32),
            pltpu.VMEM((16,), jnp.int32),
            pltpu.SemaphoreType.DMA,
            pltpu.SemaphoreType.DMA,
            pltpu.SemaphoreType.DMA,
            pltpu.SemaphoreType.DMA,
        ],
    )
    return f(hflat, ab, bsrc, bdst, counts, t)


def _mmh_body(x_ref, wt_ref, o_ref):
    o_ref[0] = lax.dot_general(x_ref[...], wt_ref[0],
                               (((1,), (1,)), ((), ())),
                               preferred_element_type=jnp.float32)


def _matmul_heads(x, wt, C, block_n=2000):
    n, d = x.shape
    grid = (n // block_n, 8)
    return pl.pallas_call(
        _mmh_body,
        grid=grid,
        in_specs=[
            pl.BlockSpec((block_n, d), lambda i, h: (i, 0)),
            pl.BlockSpec((1, C, d), lambda i, h: (h, 0, 0)),
        ],
        out_specs=pl.BlockSpec((1, block_n, C), lambda i, h: (h, i, 0)),
        out_shape=jax.ShapeDtypeStruct((8, n, C), jnp.float32),
    )(x, wt)


def _mm_body(x_ref, w_ref, o_ref):
    o_ref[...] = jnp.dot(x_ref[...], w_ref[...],
                         preferred_element_type=jnp.float32)


def _matmul(x, w, block_n=2000):
    n, d = x.shape
    f = w.shape[1]
    return pl.pallas_call(
        _mm_body,
        grid=(n // block_n,),
        in_specs=[
            pl.BlockSpec((block_n, d), lambda i: (i, 0)),
            pl.BlockSpec((d, f), lambda i: (0, 0)),
        ],
        out_specs=pl.BlockSpec((block_n, f), lambda i: (i, 0)),
        out_shape=jax.ShapeDtypeStruct((n, f), jnp.float32),
    )(x, w)


def _mid_body(z_ref, o_ref):
    o_ref[...] = 1.0 / (z_ref[0] + z_ref[1] + 1e-16)


def _mid(zp):
    z2 = zp.reshape(NSC, NPZ // 16, 256)
    rz = pl.pallas_call(
        _mid_body,
        in_specs=[pl.BlockSpec((NSC, NPZ // 16, 256), lambda: (0, 0, 0))],
        out_specs=pl.BlockSpec((NPZ // 16, 256), lambda: (0, 0)),
        out_shape=jax.ShapeDtypeStruct((NPZ // 16, 256), jnp.float32),
    )(z2)
    return rz.reshape(NPZ, 16)


def _fin_body(num_ref, den_ref, b_ref, o_ref):
    o_ref[...] = jax.nn.relu(num_ref[...] / (den_ref[...] + 1e-16)
                             + b_ref[...])


def _finalize(num, den, b, n, C, block_n=2000):
    F = 8 * C
    return pl.pallas_call(
        _fin_body,
        grid=(n // block_n,),
        in_specs=[
            pl.BlockSpec((block_n, F), lambda i: (i, 0)),
            pl.BlockSpec((block_n, F), lambda i: (i, 0)),
            pl.BlockSpec((1, F), lambda i: (0, 0)),
        ],
        out_specs=pl.BlockSpec((block_n, F), lambda i: (i, 0)),
        out_shape=jax.ShapeDtypeStruct((n, F), jnp.float32),
    )(num, den, b.reshape(1, F))


def _gat_layer(x, srcr, dstr, bsrc, bdst, bk, counts, W, att_src, att_dst,
               b, t, E):
    n = x.shape[0]
    heads, ch = att_src.shape
    d = W.shape[0]
    ws = (W.reshape(d, heads, ch) * att_src[None]).sum(-1)
    wd = (W.reshape(d, heads, ch) * att_dst[None]).sum(-1)
    wt = W.T.reshape(heads, ch, d)
    h_hm = _matmul_heads(x, wt, ch)
    a_cat = _matmul(x, jnp.concatenate([ws, wd], axis=1))
    p, zp = _phase1(a_cat, srcr, dstr, E)
    rz = _mid(zp)
    ab = _phase1c(p, rz, bdst, bk, counts, E)
    num, den = _phase2(h_hm.reshape(heads * n, ch), ab, bsrc, bdst, counts,
                       jnp.full((16,), t, jnp.float32), n, ch)
    return _finalize(num, den, b, n, ch)


def kernel(x, edge_index, W1, as1, ad1, b1, t1, W2, as2, ad2, b2, t2):
    E = edge_index.shape[1]
    srcr = edge_index[0].reshape(E // 128, 128)
    dstr = edge_index[1].reshape(E // 128, 128)
    bsrc, bdst, bk, counts = _phase0(srcr, dstr, E)
    h1 = _gat_layer(x, srcr, dstr, bsrc, bdst, bk, counts,
                    W1, as1, ad1, b1, t1, E)
    return _gat_layer(h1, srcr, dstr, bsrc, bdst, bk, counts,
                      W2, as2, ad2, b2, t2, E)


# speedup vs baseline: 3.1629x; 1.0238x over previous
import functools

import jax
import jax.numpy as jnp
from jax import lax
from jax.experimental import pallas as pl
from jax.experimental.pallas import tpu as pltpu
from jax.experimental.pallas import tpu_sc as plsc

NSC, NSUB = 2, 16
NW = NSC * NSUB
NPT = 320
NROW = NW * NPT
NPZ = 10368
ZSTRIPE = NPZ // NSUB


def _mesh():
    return plsc.VectorSubcoreMesh(core_axis_name="c", subcore_axis_name="s")


def _wid():
    return lax.axis_index("s") * NSC + lax.axis_index("c")


def _iota16():
    return lax.iota(jnp.int32, 16)


def _dyn_gather(x, idx):
    return x.at[idx].get(mode="promise_in_bounds")


def _phase0(srcr, dstr, E):
    nrows = E // 128
    SR = 20
    nchunks = nrows // SR
    ECAP = E + 128

    def body(srcr_h, dstr_h, bsrc_h, bdst_h, bk_h, cnt_h,
             sv, dv, st_s, st_d, st_k, csm):
        w = _wid()
        lo = w * NPT
        iota = _iota16()

        def chunk_body(ci, carry):
            cur, out = carry
            rbase = ci * SR
            pltpu.sync_copy(srcr_h.at[pl.ds(rbase, SR), :], sv)
            pltpu.sync_copy(dstr_h.at[pl.ds(rbase, SR), :], dv)

            def group_body(g, carry2):
                cur2, = carry2
                j = g // 8
                sub = g % 8
                d16 = dv[j, pl.ds(sub * 16, 16)]
                s16 = sv[j, pl.ds(sub * 16, 16)]
                k16 = (ci * (SR * 128) + g * 16) + iota
                msk = (d16 >= lo) & (d16 < lo + NPT)
                plsc.store_compressed(st_s.at[pl.ds(cur2, 16)], s16, mask=msk)
                plsc.store_compressed(st_d.at[pl.ds(cur2, 16)], d16 - lo,
                                      mask=msk)
                plsc.store_compressed(st_k.at[pl.ds(cur2, 16)], k16, mask=msk)
                cnt = jnp.sum(msk.astype(jnp.int32))
                return (cur2 + cnt,)

            cur, = lax.fori_loop(0, SR * 8, group_body, (cur,))

            nfl = cur // 128

            def flush_body(j, _):
                o = pl.multiple_of(out + j * 128, 128)
                pltpu.sync_copy(st_s.at[pl.ds(j * 128, 128)],
                                bsrc_h.at[w, pl.ds(o, 128)])
                pltpu.sync_copy(st_d.at[pl.ds(j * 128, 128)],
                                bdst_h.at[w, pl.ds(o, 128)])
                pltpu.sync_copy(st_k.at[pl.ds(j * 128, 128)],
                                bk_h.at[w, pl.ds(o, 128)])
                return 0

            lax.fori_loop(0, nfl, flush_body, 0)
            tb = nfl * 128
            for i in range(8):
                st_s[pl.ds(i * 16, 16)] = st_s[pl.ds(tb + i * 16, 16)]
                st_d[pl.ds(i * 16, 16)] = st_d[pl.ds(tb + i * 16, 16)]
                st_k[pl.ds(i * 16, 16)] = st_k[pl.ds(tb + i * 16, 16)]
            return (cur - tb, out + tb)

        cur, out = lax.fori_loop(0, nchunks, chunk_body, (0, 0))

        @pl.when(cur > 0)
        def _():
            for i in range(8):
                st_s[pl.ds(cur + i * 16, 16)] = jnp.zeros(16, jnp.int32)
                st_d[pl.ds(cur + i * 16, 16)] = jnp.full(16, NPT, jnp.int32)
                st_k[pl.ds(cur + i * 16, 16)] = jnp.zeros(16, jnp.int32)
            o = pl.multiple_of(out, 128)
            pltpu.sync_copy(st_s.at[pl.ds(0, 128)],
                            bsrc_h.at[w, pl.ds(o, 128)])
            pltpu.sync_copy(st_d.at[pl.ds(0, 128)],
                            bdst_h.at[w, pl.ds(o, 128)])
            pltpu.sync_copy(st_k.at[pl.ds(0, 128)],
                            bk_h.at[w, pl.ds(o, 128)])

        total = jnp.where(cur > 0, out + 128, out)
        csm[:] = jnp.full(16, total, jnp.int32)
        pltpu.sync_copy(csm, cnt_h.at[w, :])

    f = pl.kernel(
        body,
        out_type=(
            jax.ShapeDtypeStruct((NW, ECAP), jnp.int32),
            jax.ShapeDtypeStruct((NW, ECAP), jnp.int32),
            jax.ShapeDtypeStruct((NW, ECAP), jnp.int32),
            jax.ShapeDtypeStruct((NW, 16), jnp.int32),
        ),
        mesh=_mesh(),
        compiler_params=pltpu.CompilerParams(use_tc_tiling_on_sc=False, needs_layout_passes=False),
        scratch_types=[
            pltpu.VMEM((SR, 128), jnp.int32),
            pltpu.VMEM((SR, 128), jnp.int32),
            pltpu.VMEM((2704,), jnp.int32),
            pltpu.VMEM((2704,), jnp.int32),
            pltpu.VMEM((2704,), jnp.int32),
            pltpu.VMEM((16,), jnp.int32),
        ],
    )
    return f(srcr, dstr)


def _phase1(a_cat, srcr, dstr, E):
    NBLK = E // 512
    npw = (NBLK + NW - 1) // NW

    def body(a_h, srcr_h, dstr_h, p_h, zp_h,
             sv, dv, as_r, ad_r, pb, zb, z_sh):
        w = _wid()
        cid = lax.axis_index("c")
        sid = lax.axis_index("s")
        iota = _iota16()

        def zp_body(r, _):
            pb[r, :] = jnp.zeros(16, jnp.float32)
            return 0
        lax.fori_loop(0, 512, zp_body, 0)

        def zz_body(r, _):
            zb[r, :] = jnp.zeros(16, jnp.float32)
            return 0
        lax.fori_loop(0, ZSTRIPE, zz_body, 0)
        pltpu.sync_copy(zb, z_sh.at[pl.ds(pl.multiple_of(sid * ZSTRIPE, 8),
                                          ZSTRIPE), :])
        plsc.subcore_barrier()

        def blk_body(j, _):
            b = j * NW + w

            @pl.when(b < NBLK)
            def _():
                rbase = b * 4
                pltpu.sync_copy(srcr_h.at[pl.ds(rbase, 4), :], sv)
                pltpu.sync_copy(dstr_h.at[pl.ds(rbase, 4), :], dv)
                for jj in range(4):
                    pltpu.sync_copy(a_h.at[sv.at[jj]],
                                    as_r.at[pl.ds(jj * 128, 128), :])
                    pltpu.sync_copy(a_h.at[dv.at[jj]],
                                    ad_r.at[pl.ds(jj * 128, 128), :])

                def grp_body(g, _2):
                    e16 = g * 16 + iota
                    fhs = [jnp.full(16, h, jnp.int32) for h in range(8)]
                    avs = [plsc.load_gather(as_r, [e16, fh]) for fh in fhs]
                    bvs = [plsc.load_gather(ad_r, [e16, fh + 8])
                           for fh in fhs]
                    evs = [a + b for a, b in zip(avs, bvs)]
                    lvs = [jnp.maximum(ev, 0.0) + 0.2 * jnp.minimum(ev, 0.0)
                           for ev in evs]
                    pvs = [jnp.exp(lv) for lv in lvs]
                    for h in range(8):
                        plsc.store_scatter(pb, [e16, fhs[h]], pvs[h])
                    return 0

                lax.fori_loop(0, 32, grp_body, 0)
                pltpu.sync_copy(pb, p_h.at[pl.ds(pl.multiple_of(b * 512, 512), 512), :])
                for jj in range(4):
                    pltpu.sync_copy(pb.at[pl.ds(jj * 128, 128), :],
                                    z_sh.at[dv.at[jj]], add=True)
            return 0

        lax.fori_loop(0, npw, blk_body, 0)
        plsc.subcore_barrier()
        st8 = pl.multiple_of(sid * ZSTRIPE, 8)
        pltpu.sync_copy(z_sh.at[pl.ds(st8, ZSTRIPE), :],
                        zp_h.at[cid, pl.ds(st8, ZSTRIPE), :])

    f = pl.kernel(
        body,
        out_type=(
            jax.ShapeDtypeStruct((E, 16), jnp.float32),
            jax.ShapeDtypeStruct((NSC, NPZ, 16), jnp.float32),
        ),
        mesh=_mesh(),
        compiler_params=pltpu.CompilerParams(use_tc_tiling_on_sc=False, needs_layout_passes=False),
        scratch_types=[
            pltpu.VMEM((4, 128), jnp.int32),
            pltpu.VMEM((4, 128), jnp.int32),
            pltpu.VMEM((512, 16), jnp.float32),
            pltpu.VMEM((512, 16), jnp.float32),
            pltpu.VMEM((512, 16), jnp.float32),
            pltpu.VMEM((ZSTRIPE, 16), jnp.float32),
            pltpu.VMEM_SHARED((NPZ, 16), jnp.float32),
        ],
    )
    return f(a_cat, srcr, dstr)


def _phase1c(p, rz, bdst, bk, counts, E):
    ECAP = E + 128

    def body(p_h, rz_h, bdst_h, bk_h, cnt_h, ab_h,
             kb, dlb, dgb, prows, zrows, abuf, csm):
        w = _wid()
        iota = _iota16()
        pltpu.sync_copy(cnt_h.at[w, :], csm)
        nb = jnp.max(csm[...]) // 128

        def bat_body(jb, _):
            off = pl.multiple_of(jb * 128, 128)
            pltpu.sync_copy(bk_h.at[w, pl.ds(off, 128)], kb)
            pltpu.sync_copy(bdst_h.at[w, pl.ds(off, 128)], dlb)
            for i in range(8):
                dgb[pl.ds(i * 16, 16)] = dlb[pl.ds(i * 16, 16)] + w * NPT
            pltpu.sync_copy(p_h.at[kb], prows)
            pltpu.sync_copy(rz_h.at[dgb], zrows)
            for i in range(8):
                e16 = i * 16 + iota
                for h in range(8):
                    fh = jnp.full(16, h, jnp.int32)
                    av = (plsc.load_gather(prows, [e16, fh])
                          * plsc.load_gather(zrows, [e16, fh]))
                    plsc.store_scatter(abuf, [e16, fh], av)
            pltpu.sync_copy(abuf, ab_h.at[w, pl.ds(off, 128), :])
            return 0

        lax.fori_loop(0, nb, bat_body, 0)

    f = pl.kernel(
        body,
        out_type=jax.ShapeDtypeStruct((NW, ECAP, 8), jnp.float32),
        mesh=_mesh(),
        compiler_params=pltpu.CompilerParams(use_tc_tiling_on_sc=False, needs_layout_passes=False),
        scratch_types=[
            pltpu.VMEM((128,), jnp.int32),
            pltpu.VMEM((128,), jnp.int32),
            pltpu.VMEM((128,), jnp.int32),
            pltpu.VMEM((128, 16), jnp.float32),
            pltpu.VMEM((128, 16), jnp.float32),
            pltpu.VMEM((128, 8), jnp.float32),
            pltpu.VMEM((16,), jnp.int32),
        ],
    )
    return f(p, rz, bdst, bk, counts)


def _phase2(hflat, ab, bsrc, bdst, counts, t, n, C):
    def body(h_h, ab_h, bsrc_h, bdst_h, cnt_h, t_h, num_h, den_h,
             srcv, srcv2, hrows, nacc, dacc, dst_vm, al_vm, tv, csm,
             sema0, sema1, semh0, semh1):
        w = _wid()
        pltpu.sync_copy(cnt_h.at[w, :], csm)
        pltpu.sync_copy(t_h, tv)
        nb = jnp.max(csm[...]) // 128
        tl2 = tv[...]
        NC16 = C // 16
        sema = [sema0, sema1]
        semh = [semh0, semh1]

        def meta_refs(jb, s):
            off = pl.multiple_of(jb * 128, 128)
            return [(bsrc_h.at[w, pl.ds(off, 128)], srcv.at[s]),
                    (bdst_h.at[w, pl.ds(off, 128)], dst_vm.at[s]),
                    (ab_h.at[w, pl.ds(off, 128), :], al_vm.at[s])]

        def issue_meta(jb, s):
            for a, b in meta_refs(jb, s):
                pltpu.async_copy(a, b, sema[s])

        def wait_meta(jb, s):
            for a, b in meta_refs(jb, s):
                pltpu.make_async_copy(a, b, sema[s]).wait()

        def issue_h(s, head):
            for i in range(8):
                srcv2[s, pl.ds(i * 16, 16)] = (srcv[s, pl.ds(i * 16, 16)]
                                               + head * n)
            pltpu.async_copy(h_h.at[srcv2.at[s]], hrows.at[s], semh[s])

        def wait_h(s):
            pltpu.make_async_copy(h_h.at[srcv2.at[s]], hrows.at[s],
                                  semh[s]).wait()

        def chunk_body(head, _):
            def zb(r, _2):
                for c in range(NC16):
                    nacc[r, pl.ds(c * 16, 16)] = jnp.zeros(16, jnp.float32)
                    dacc[r, pl.ds(c * 16, 16)] = jnp.zeros(16, jnp.float32)
                return 0
            lax.fori_loop(0, NPT + 1, zb, 0)
            iota = _iota16()
            fh = jnp.full(16, 0, jnp.int32) + head

            def compute(s):
                def grp_body(gi, _3):
                    gb = gi * 16
                    d16 = dst_vm[s, pl.ds(gb, 16)]
                    av16 = plsc.load_gather(al_vm.at[s], [gb + iota, fh])
                    for le in range(16):
                        sel = jnp.full((16,), le, jnp.int32)
                        al = _dyn_gather(av16, sel)
                        dspl = _dyn_gather(d16, sel)
                        hvs = [hrows[s, gb + le, pl.ds(c * 16, 16)]
                               for c in range(NC16)]
                        us = [hv * al for hv in hvs]
                        gs = [jnp.exp(u * tl2) for u in us]
                        qs = [g * u for g, u in zip(gs, us)]
                        for c in range(NC16):
                            colv = c * 16 + iota
                            plsc.addupdate_scatter(dacc, [dspl, colv], gs[c])
                            plsc.addupdate_scatter(nacc, [dspl, colv], qs[c])
                    return 0

                lax.fori_loop(0, 8, grp_body, 0)

            @pl.when(nb > 0)
            def _():
                issue_meta(0, 0)
                wait_meta(0, 0)
                issue_h(0, head)

            @pl.when(nb > 1)
            def _():
                issue_meta(1, 1)

            def pair_body(jp, _2):
                jb0 = jp * 2
                jb1 = jb0 + 1

                @pl.when(jb1 < nb)
                def _():
                    wait_meta(jb1, 1)
                    issue_h(1, head)
                wait_h(0)
                compute(0)

                @pl.when(jb0 + 2 < nb)
                def _():
                    issue_meta(jb0 + 2, 0)

                @pl.when(jb1 + 1 < nb)
                def _():
                    wait_meta(jb1 + 1, 0)
                    issue_h(0, head)
                wait_h(1)
                compute(1)

                @pl.when(jb1 + 2 < nb)
                def _():
                    issue_meta(jb1 + 2, 1)
                return 0

            lax.fori_loop(0, nb // 2, pair_body, 0)

            @pl.when((nb % 2) == 1)
            def _():
                wait_h(0)
                compute(0)

            hc = pl.multiple_of(head * C, C)
            rw = pl.multiple_of(w * NPT, NPT)
            pltpu.sync_copy(nacc.at[pl.ds(0, NPT), :],
                            num_h.at[pl.ds(rw, NPT), pl.ds(hc, C)])
            pltpu.sync_copy(dacc.at[pl.ds(0, NPT), :],
                            den_h.at[pl.ds(rw, NPT), pl.ds(hc, C)])
            return 0

        lax.fori_loop(0, 8, chunk_body, 0)

    f = pl.kernel(
        body,
        out_type=(
            jax.ShapeDtypeStruct((NROW, 8 * C), jnp.float32),
            jax.ShapeDtypeStruct((NROW, 8 * C), jnp.float32),
        ),
        mesh=_mesh(),
        compiler_params=pltpu.CompilerParams(use_tc_tiling_on_sc=False, needs_layout_passes=False),
        scratch_types=[
            pltpu.VMEM((2, 128), jnp.int32),
            pltpu.VMEM((2, 128), jnp.int32),
            pltpu.VMEM((2, 128, C), jnp.float32),
            pltpu.VMEM((NPT + 1, C), jnp.float32),
            pltpu.VMEM((NPT + 1, C), jnp.float32),
            pltpu.VMEM((2, 128), jnp.int32),
            pltpu.VMEM((2, 128, 8), jnp.float32),
            pltpu.VMEM((16,), jnp.float32),
            pltpu.VMEM((16,), jnp.int32),
            pltpu.SemaphoreType.DMA,
            pltpu.SemaphoreType.DMA,
            pltpu.SemaphoreType.DMA,
            pltpu.SemaphoreType.DMA,
        ],
    )
    return f(hflat, ab, bsrc, bdst, counts, t)


def _mmh_body(x_ref, wt_ref, o_ref):
    o_ref[0] = lax.dot_general(x_ref[...], wt_ref[0],
                               (((1,), (1,)), ((), ())),
                               preferred_element_type=jnp.float32)


def _matmul_heads(x, wt, C, block_n=2000):
    n, d = x.shape
    grid = (n // block_n, 8)
    return pl.pallas_call(
        _mmh_body,
        grid=grid,
        in_specs=[
            pl.BlockSpec((block_n, d), lambda i, h: (i, 0)),
            pl.BlockSpec((1, C, d), lambda i, h: (h, 0, 0)),
        ],
        out_specs=pl.BlockSpec((1, block_n, C), lambda i, h: (h, i, 0)),
        out_shape=jax.ShapeDtypeStruct((8, n, C), jnp.float32),
    )(x, wt)


def _mm_body(x_ref, w_ref, o_ref):
    o_ref[...] = jnp.dot(x_ref[...], w_ref[...],
                         preferred_element_type=jnp.float32)


def _matmul(x, w, block_n=2000):
    n, d = x.shape
    f = w.shape[1]
    return pl.pallas_call(
        _mm_body,
        grid=(n // block_n,),
        in_specs=[
            pl.BlockSpec((block_n, d), lambda i: (i, 0)),
            pl.BlockSpec((d, f), lambda i: (0, 0)),
        ],
        out_specs=pl.BlockSpec((block_n, f), lambda i: (i, 0)),
        out_shape=jax.ShapeDtypeStruct((n, f), jnp.float32),
    )(x, w)


def _mid_body(z_ref, o_ref):
    o_ref[...] = 1.0 / (z_ref[0] + z_ref[1] + 1e-16)


def _mid(zp):
    z2 = zp.reshape(NSC, NPZ // 16, 256)
    rz = pl.pallas_call(
        _mid_body,
        in_specs=[pl.BlockSpec((NSC, NPZ // 16, 256), lambda: (0, 0, 0))],
        out_specs=pl.BlockSpec((NPZ // 16, 256), lambda: (0, 0)),
        out_shape=jax.ShapeDtypeStruct((NPZ // 16, 256), jnp.float32),
    )(z2)
    return rz.reshape(NPZ, 16)


def _fin_body(num_ref, den_ref, b_ref, o_ref):
    o_ref[...] = jax.nn.relu(num_ref[...] / (den_ref[...] + 1e-16)
                             + b_ref[...])


def _finalize(num, den, b, n, C, block_n=2000):
    F = 8 * C
    return pl.pallas_call(
        _fin_body,
        grid=(n // block_n,),
        in_specs=[
            pl.BlockSpec((block_n, F), lambda i: (i, 0)),
            pl.BlockSpec((block_n, F), lambda i: (i, 0)),
            pl.BlockSpec((1, F), lambda i: (0, 0)),
        ],
        out_specs=pl.BlockSpec((block_n, F), lambda i: (i, 0)),
        out_shape=jax.ShapeDtypeStruct((n, F), jnp.float32),
    )(num, den, b.reshape(1, F))


def _gat_layer(x, srcr, dstr, bsrc, bdst, bk, counts, W, att_src, att_dst,
               b, t, E):
    n = x.shape[0]
    heads, ch = att_src.shape
    d = W.shape[0]
    ws = (W.reshape(d, heads, ch) * att_src[None]).sum(-1)
    wd = (W.reshape(d, heads, ch) * att_dst[None]).sum(-1)
    wt = W.T.reshape(heads, ch, d)
    h_hm = _matmul_heads(x, wt, ch)
    a_cat = _matmul(x, jnp.concatenate([ws, wd], axis=1))
    p, zp = _phase1(a_cat, srcr, dstr, E)
    rz = _mid(zp)
    ab = _phase1c(p, rz, bdst, bk, counts, E)
    num, den = _phase2(h_hm.reshape(heads * n, ch), ab, bsrc, bdst, counts,
                       jnp.full((16,), t, jnp.float32), n, ch)
    return _finalize(num, den, b, n, ch)


def kernel(x, edge_index, W1, as1, ad1, b1, t1, W2, as2, ad2, b2, t2):
    E = edge_index.shape[1]
    srcr = edge_index[0].reshape(E // 128, 128)
    dstr = edge_index[1].reshape(E // 128, 128)
    bsrc, bdst, bk, counts = _phase0(srcr, dstr, E)
    h1 = _gat_layer(x, srcr, dstr, bsrc, bdst, bk, counts,
                    W1, as1, ad1, b1, t1, E)
    return _gat_layer(h1, srcr, dstr, bsrc, bdst, bk, counts,
                      W2, as2, ad2, b2, t2, E)
